# Initial kernel scaffold; baseline (speedup 1.0000x reference)
import jax
import jax.numpy as jnp
from jax.experimental import pallas as pl


def kernel(h, phrase_mask, phrase_token_idx, phrase_end_pos, rope_cos, rope_sin, W_dq, W_uq, W_kv_kv, W_z_kv, B_pos_kv, W_kv_idx, W_z_idx, B_pos_idx, W_iuq, W_w, q_norm_w, k_norm_w, W_o, sink_logits):
    raise NotImplementedError("write your pallas kernel here")


# speedup vs baseline: 11.1517x; 11.1517x over previous
import math

import jax
import jax.numpy as jnp
from jax.experimental import pallas as pl

NEG = -1e30
NEG2 = -2e30


def _bdot(a, b, dims=None):
    a = a.astype(jnp.bfloat16)
    b = b.astype(jnp.bfloat16)
    if dims is None:
        return jnp.dot(a, b, preferred_element_type=jnp.float32)
    return jax.lax.dot_general(a, b, dims, preferred_element_type=jnp.float32)


def _proj_body(h_ref, wc_ref, wdq_ref, wwp_ref, pc_ref, ql_ref, wh_ref):
    hb = h_ref[0]
    pc_ref[0] = _bdot(hb, wc_ref[...].T)
    ql_ref[0] = _bdot(hb, wdq_ref[...].T)
    wh_ref[0] = _bdot(hb, wwp_ref[...].T)


def _compress_body(pc_ref, idx_ref, ep_ref, bkv_ref, bidx_ref, cos_ref, sin_ref,
                   knw_ref, ccomp_ref, kidx_ref, kall_ref, *, C, LMAX, PB, T,
                   TCHUNK):
    idx2 = idx_ref[0, 0, 0][None, :]
    n = PB * LMAX
    g = jnp.zeros((n, 4 * C), jnp.float32)
    for k in range(T // TCHUNK):
        rows = jax.lax.broadcasted_iota(jnp.int32, (TCHUNK, n), 0) + k * TCHUNK
        oh = (rows == idx2).astype(jnp.float32)
        chunk = pc_ref[0, k * TCHUNK:(k + 1) * TCHUNK, :]
        g = g + jax.lax.dot_general(
            oh, chunk, (((0,), (0,)), ((), ())),
            preferred_element_type=jnp.float32, precision=jax.lax.Precision.HIGHEST)
    g3 = g.reshape(PB, LMAX, 4 * C)

    def compress(ctok, z):
        m = jnp.max(z, axis=1, keepdims=True)
        e = jnp.exp(z - m)
        gates = e / jnp.sum(e, axis=1, keepdims=True)
        return jnp.sum(gates * ctok, axis=1)

    ccomp = compress(g3[:, :, 0:C], g3[:, :, C:2 * C] + bkv_ref[...][None])
    kidx = compress(g3[:, :, 2 * C:3 * C],
                    g3[:, :, 3 * C:4 * C] + bidx_ref[...][None])
    ccomp_ref[0] = ccomp
    kidx_ref[0] = kidx

    ms = jnp.mean(ccomp * ccomp, axis=1, keepdims=True)
    kn = ccomp * jax.lax.rsqrt(ms + 1e-6) * knw_ref[...]
    ep = ep_ref[0, 0]
    prow = jax.lax.broadcasted_iota(jnp.int32, (T, PB), 0)
    ohp = (prow == ep).astype(jnp.float32)
    cos_k = jax.lax.dot_general(ohp, cos_ref[...], (((0,), (0,)), ((), ())),
                                preferred_element_type=jnp.float32, precision=jax.lax.Precision.HIGHEST)
    sin_k = jax.lax.dot_general(ohp, sin_ref[...], (((0,), (0,)), ((), ())),
                                preferred_element_type=jnp.float32, precision=jax.lax.Precision.HIGHEST)
    half = C // 2
    krot = jnp.concatenate([-kn[:, half:], kn[:, :half]], axis=1)
    kall_ref[0] = kn * cos_k + krot * sin_k


def _attn_body(ql_ref, wh_ref, kidx_ref, ccomp_ref, kall_ref, ep_ref,
               cos_ref, sin_ref, wuq_ref, wiuq_ref, wo_ref, qnw_ref,
               sinkp_ref, out_ref, *, C, H, NIH, P, TT, TOP_K):
    j = pl.program_id(1)
    ql = ql_ref[0]
    kidx = kidx_ref[0]
    ccomp = ccomp_ref[0]
    kall = kall_ref[0]
    ep = ep_ref[0]
    pos = jax.lax.broadcasted_iota(jnp.int32, (TT, 1), 0) + j * TT
    vis = ep < pos

    qi = _bdot(ql, wiuq_ref[...].T)
    wh = wh_ref[0]
    scores = jnp.zeros((TT, P), jnp.float32)
    for ih in range(NIH):
        qih = qi[:, ih * C:(ih + 1) * C]
        s = _bdot(qih, kidx.T)
        scores = scores + jnp.maximum(s, 0.0) * wh[:, ih:ih + 1]
    scores = jnp.where(vis, scores, NEG)

    lane = jax.lax.broadcasted_iota(jnp.int32, (TT, P), 1)
    sel = jnp.zeros((TT, P), jnp.bool_)
    avail = jnp.ones((TT, P), jnp.bool_)
    for _ in range(TOP_K):
        cur = jnp.where(avail, scores, NEG2)
        m = jnp.max(cur, axis=1, keepdims=True)
        cand = jnp.where(cur == m, lane, P)
        j0 = jnp.min(cand, axis=1, keepdims=True)
        pick = lane == j0
        sel = sel | pick
        avail = avail & jnp.logical_not(pick)
    amask = sel & vis
    has_any = jnp.sum(vis.astype(jnp.float32), axis=1, keepdims=True) > 0.0

    qfull = _bdot(ql, wuq_ref[...].T)
    cos_q = cos_ref[...]
    sin_q = sin_ref[...]
    half = C // 2
    qnw = qnw_ref[...]
    inv_sqrt_c = 1.0 / math.sqrt(C)
    outs = []
    for hh in range(H):
        qh = qfull[:, hh * C:(hh + 1) * C]
        ms = jnp.mean(qh * qh, axis=1, keepdims=True)
        qh = qh * jax.lax.rsqrt(ms + 1e-6) * qnw
        qrot = jnp.concatenate([-qh[:, half:], qh[:, :half]], axis=1)
        qh = qh * cos_q + qrot * sin_q
        lg = _bdot(qh, kall.T)
        lg = lg * inv_sqrt_c
        lg = jnp.where(amask, lg, NEG)
        sk = sinkp_ref[:, hh:hh + 1]
        m = jnp.maximum(jnp.max(lg, axis=1, keepdims=True), sk)
        e = jnp.exp(lg - m)
        denom = jnp.sum(e, axis=1, keepdims=True) + jnp.exp(sk - m)
        attn = e / denom
        attn = jnp.where(has_any, attn, 0.0)
        outs.append(_bdot(attn, ccomp))
    ofull = jnp.concatenate(outs, axis=1)
    out_ref[0] = _bdot(ofull, wo_ref[...].T)


def _phases(h, phrase_mask, phrase_token_idx, phrase_end_pos, rope_cos,
            rope_sin, W_dq, W_uq, W_kv_kv, W_z_kv, B_pos_kv, W_kv_idx,
            W_z_idx, B_pos_idx, W_iuq, W_w, q_norm_w, k_norm_w, W_o,
            sink_logits):
    B, T, D = h.shape
    _, P, LMAX = phrase_token_idx.shape
    C = W_kv_kv.shape[0]
    QCD = W_dq.shape[0]
    H = sink_logits.shape[0]
    IHD = W_kv_idx.shape[0]
    NIH = W_iuq.shape[0] // IHD
    TOP_K = min(32, P)
    NW = W_w.shape[0]

    TQ = min(512, T)
    PB = min(64, P)
    TT = min(256, T)
    TCHUNK = min(1024, T)

    Wc = jnp.concatenate([W_kv_kv, W_z_kv, W_kv_idx, W_z_idx], axis=0)
    WHP = 128
    W_wp = jnp.zeros((WHP, D), jnp.float32).at[:NW].set(W_w)
    idx4 = phrase_token_idx.reshape(B, P // PB, 1, PB * LMAX).astype(jnp.int32)
    ep4 = phrase_end_pos.reshape(B, P // PB, 1, PB).astype(jnp.int32)
    ep3 = phrase_end_pos.reshape(B, 1, P).astype(jnp.int32)
    qnw2 = q_norm_w.reshape(1, C)
    knw2 = k_norm_w.reshape(1, C)
    sinkp = jnp.zeros((1, 128), jnp.float32).at[0, :H].set(sink_logits)

    pc, ql, wh = pl.pallas_call(
        _proj_body,
        grid=(B, T // TQ),
        in_specs=[
            pl.BlockSpec((1, TQ, D), lambda b, i: (b, i, 0)),
            pl.BlockSpec((4 * C, D), lambda b, i: (0, 0)),
            pl.BlockSpec((QCD, D), lambda b, i: (0, 0)),
            pl.BlockSpec((WHP, D), lambda b, i: (0, 0)),
        ],
        out_specs=[
            pl.BlockSpec((1, TQ, 4 * C), lambda b, i: (b, i, 0)),
            pl.BlockSpec((1, TQ, QCD), lambda b, i: (b, i, 0)),
            pl.BlockSpec((1, TQ, WHP), lambda b, i: (b, i, 0)),
        ],
        out_shape=[
            jax.ShapeDtypeStruct((B, T, 4 * C), jnp.float32),
            jax.ShapeDtypeStruct((B, T, QCD), jnp.float32),
            jax.ShapeDtypeStruct((B, T, WHP), jnp.float32),
        ],
    )(h, Wc, W_dq, W_wp)

    from functools import partial
    c_comp, k_idx, k_all = pl.pallas_call(
        partial(_compress_body, C=C, LMAX=LMAX, PB=PB, T=T, TCHUNK=TCHUNK),
        grid=(B, P // PB),
        in_specs=[
            pl.BlockSpec((1, T, 4 * C), lambda b, p: (b, 0, 0)),
            pl.BlockSpec((1, 1, 1, PB * LMAX), lambda b, p: (b, p, 0, 0)),
            pl.BlockSpec((1, 1, 1, PB), lambda b, p: (b, p, 0, 0)),
            pl.BlockSpec((LMAX, C), lambda b, p: (0, 0)),
            pl.BlockSpec((LMAX, C), lambda b, p: (0, 0)),
            pl.BlockSpec((T, C), lambda b, p: (0, 0)),
            pl.BlockSpec((T, C), lambda b, p: (0, 0)),
            pl.BlockSpec((1, C), lambda b, p: (0, 0)),
        ],
        out_specs=[
            pl.BlockSpec((1, PB, C), lambda b, p: (b, p, 0)),
            pl.BlockSpec((1, PB, C), lambda b, p: (b, p, 0)),
            pl.BlockSpec((1, PB, C), lambda b, p: (b, p, 0)),
        ],
        out_shape=[
            jax.ShapeDtypeStruct((B, P, C), jnp.float32),
            jax.ShapeDtypeStruct((B, P, C), jnp.float32),
            jax.ShapeDtypeStruct((B, P, C), jnp.float32),
        ],
    )(pc, idx4, ep4, B_pos_kv, B_pos_idx, rope_cos, rope_sin, knw2)

    out = pl.pallas_call(
        partial(_attn_body, C=C, H=H, NIH=NIH, P=P, TT=TT, TOP_K=TOP_K),
        grid=(B, T // TT),
        in_specs=[
            pl.BlockSpec((1, TT, QCD), lambda b, i: (b, i, 0)),
            pl.BlockSpec((1, TT, WHP), lambda b, i: (b, i, 0)),
            pl.BlockSpec((1, P, C), lambda b, i: (b, 0, 0)),
            pl.BlockSpec((1, P, C), lambda b, i: (b, 0, 0)),
            pl.BlockSpec((1, P, C), lambda b, i: (b, 0, 0)),
            pl.BlockSpec((1, 1, P), lambda b, i: (b, 0, 0)),
            pl.BlockSpec((TT, C), lambda b, i: (i, 0)),
            pl.BlockSpec((TT, C), lambda b, i: (i, 0)),
            pl.BlockSpec((H * C, QCD), lambda b, i: (0, 0)),
            pl.BlockSpec((NIH * IHD, QCD), lambda b, i: (0, 0)),
            pl.BlockSpec((D, H * C), lambda b, i: (0, 0)),
            pl.BlockSpec((1, C), lambda b, i: (0, 0)),
            pl.BlockSpec((1, 128), lambda b, i: (0, 0)),
        ],
        out_specs=pl.BlockSpec((1, TT, D), lambda b, i: (b, i, 0)),
        out_shape=jax.ShapeDtypeStruct((B, T, D), jnp.float32),
    )(ql, wh, k_idx, c_comp, k_all, ep3, rope_cos, rope_sin, W_uq, W_iuq,
      W_o, qnw2, sinkp)
    return out, (pc, ql, wh, c_comp, k_idx, k_all)


def kernel(*args):
    return _phases(*args)[0]


# speedup vs baseline: 13.2285x; 1.1862x over previous
import math

import jax
import jax.numpy as jnp
from jax.experimental import pallas as pl

NEG = -1e30
NEG2 = -2e30


def _bdot(a, b, dims=None):
    a = a.astype(jnp.bfloat16)
    b = b.astype(jnp.bfloat16)
    if dims is None:
        return jnp.dot(a, b, preferred_element_type=jnp.float32)
    return jax.lax.dot_general(a, b, dims, preferred_element_type=jnp.float32)


def _proj_body(h_ref, wc_ref, wdq_ref, wwp_ref, pc_ref, ql_ref, wh_ref):
    hb = h_ref[0]
    pc_ref[0] = _bdot(hb, wc_ref[...].T)
    ql_ref[0] = _bdot(hb, wdq_ref[...].T)
    wh_ref[0] = _bdot(hb, wwp_ref[...].T)


def _compress_body(pc_ref, idx_ref, ep_ref, bkv_ref, bidx_ref, cos_ref, sin_ref,
                   knw_ref, ccomp_ref, kidx_ref, kall_ref, *, C, LMAX, PB, T,
                   TCHUNK):
    idx2 = idx_ref[0, 0, 0][None, :]
    n = PB * LMAX
    g = jnp.zeros((n, 4 * C), jnp.float32)
    for k in range(T // TCHUNK):
        rows = jax.lax.broadcasted_iota(jnp.int32, (TCHUNK, n), 0) + k * TCHUNK
        oh = (rows == idx2).astype(jnp.float32)
        chunk = pc_ref[0, k * TCHUNK:(k + 1) * TCHUNK, :]
        g = g + jax.lax.dot_general(
            oh, chunk, (((0,), (0,)), ((), ())),
            preferred_element_type=jnp.float32, precision=jax.lax.Precision.HIGHEST)
    g3 = g.reshape(PB, LMAX, 4 * C)

    def compress(ctok, z):
        m = jnp.max(z, axis=1, keepdims=True)
        e = jnp.exp(z - m)
        gates = e / jnp.sum(e, axis=1, keepdims=True)
        return jnp.sum(gates * ctok, axis=1)

    ccomp = compress(g3[:, :, 0:C], g3[:, :, C:2 * C] + bkv_ref[...][None])
    kidx = compress(g3[:, :, 2 * C:3 * C],
                    g3[:, :, 3 * C:4 * C] + bidx_ref[...][None])
    ccomp_ref[0] = ccomp
    kidx_ref[0] = kidx

    ms = jnp.mean(ccomp * ccomp, axis=1, keepdims=True)
    kn = ccomp * jax.lax.rsqrt(ms + 1e-6) * knw_ref[...]
    ep = ep_ref[0, 0]
    prow = jax.lax.broadcasted_iota(jnp.int32, (T, PB), 0)
    ohp = (prow == ep).astype(jnp.float32)
    cos_k = jax.lax.dot_general(ohp, cos_ref[...], (((0,), (0,)), ((), ())),
                                preferred_element_type=jnp.float32, precision=jax.lax.Precision.HIGHEST)
    sin_k = jax.lax.dot_general(ohp, sin_ref[...], (((0,), (0,)), ((), ())),
                                preferred_element_type=jnp.float32, precision=jax.lax.Precision.HIGHEST)
    half = C // 2
    krot = jnp.concatenate([-kn[:, half:], kn[:, :half]], axis=1)
    kall_ref[0] = kn * cos_k + krot * sin_k


def _attn_body(ql_ref, wh_ref, kidx_ref, ccomp_ref, kall_ref, ep_ref,
               cos_ref, sin_ref, wuq_ref, wiuq_ref, wo_ref, qnw_ref,
               sinkp_ref, out_ref, *, C, H, NIH, P, TT, TOP_K):
    j = pl.program_id(1)
    ql = ql_ref[0]
    kidx = kidx_ref[0]
    ccomp = ccomp_ref[0]
    kall = kall_ref[0]
    ep = ep_ref[0]
    pos = jax.lax.broadcasted_iota(jnp.int32, (TT, 1), 0) + j * TT
    vis = ep < pos

    qi = _bdot(ql, wiuq_ref[...].T)
    wh = wh_ref[0]
    scores = jnp.zeros((TT, P), jnp.float32)
    for ih in range(NIH):
        qih = qi[:, ih * C:(ih + 1) * C]
        s = _bdot(qih, kidx.T)
        scores = scores + jnp.maximum(s, 0.0) * wh[:, ih:ih + 1]
    scores = jnp.where(vis, scores, NEG)

    lane = jax.lax.broadcasted_iota(jnp.int32, (TT, P), 1)
    work = scores
    for _ in range(TOP_K):
        m = jnp.max(work, axis=1, keepdims=True)
        j0 = jnp.min(jnp.where(work == m, lane, P), axis=1, keepdims=True)
        work = jnp.where(lane == j0, NEG2, work)
    amask = (work == NEG2) & vis
    has_any = jnp.sum(vis.astype(jnp.float32), axis=1, keepdims=True) > 0.0

    qfull = _bdot(ql, wuq_ref[...].T)
    cos_q = cos_ref[...]
    sin_q = sin_ref[...]
    half = C // 2
    qnw = qnw_ref[...]
    inv_sqrt_c = 1.0 / math.sqrt(C)
    outs = []
    for hh in range(H):
        qh = qfull[:, hh * C:(hh + 1) * C]
        ms = jnp.mean(qh * qh, axis=1, keepdims=True)
        qh = qh * jax.lax.rsqrt(ms + 1e-6) * qnw
        qrot = jnp.concatenate([-qh[:, half:], qh[:, :half]], axis=1)
        qh = qh * cos_q + qrot * sin_q
        lg = _bdot(qh, kall.T)
        lg = lg * inv_sqrt_c
        lg = jnp.where(amask, lg, NEG)
        sk = sinkp_ref[:, hh:hh + 1]
        m = jnp.maximum(jnp.max(lg, axis=1, keepdims=True), sk)
        e = jnp.exp(lg - m)
        denom = jnp.sum(e, axis=1, keepdims=True) + jnp.exp(sk - m)
        attn = e / denom
        attn = jnp.where(has_any, attn, 0.0)
        outs.append(_bdot(attn, ccomp))
    ofull = jnp.concatenate(outs, axis=1)
    out_ref[0] = _bdot(ofull, wo_ref[...].T)


def _phases(h, phrase_mask, phrase_token_idx, phrase_end_pos, rope_cos,
            rope_sin, W_dq, W_uq, W_kv_kv, W_z_kv, B_pos_kv, W_kv_idx,
            W_z_idx, B_pos_idx, W_iuq, W_w, q_norm_w, k_norm_w, W_o,
            sink_logits):
    B, T, D = h.shape
    _, P, LMAX = phrase_token_idx.shape
    C = W_kv_kv.shape[0]
    QCD = W_dq.shape[0]
    H = sink_logits.shape[0]
    IHD = W_kv_idx.shape[0]
    NIH = W_iuq.shape[0] // IHD
    TOP_K = min(32, P)
    NW = W_w.shape[0]

    TQ = min(512, T)
    PB = min(64, P)
    TT = min(256, T)
    TCHUNK = min(1024, T)

    Wc = jnp.concatenate([W_kv_kv, W_z_kv, W_kv_idx, W_z_idx], axis=0)
    WHP = 128
    W_wp = jnp.zeros((WHP, D), jnp.float32).at[:NW].set(W_w)
    idx4 = phrase_token_idx.reshape(B, P // PB, 1, PB * LMAX).astype(jnp.int32)
    ep4 = phrase_end_pos.reshape(B, P // PB, 1, PB).astype(jnp.int32)
    ep3 = phrase_end_pos.reshape(B, 1, P).astype(jnp.int32)
    qnw2 = q_norm_w.reshape(1, C)
    knw2 = k_norm_w.reshape(1, C)
    sinkp = jnp.zeros((1, 128), jnp.float32).at[0, :H].set(sink_logits)

    pc, ql, wh = pl.pallas_call(
        _proj_body,
        grid=(B, T // TQ),
        in_specs=[
            pl.BlockSpec((1, TQ, D), lambda b, i: (b, i, 0)),
            pl.BlockSpec((4 * C, D), lambda b, i: (0, 0)),
            pl.BlockSpec((QCD, D), lambda b, i: (0, 0)),
            pl.BlockSpec((WHP, D), lambda b, i: (0, 0)),
        ],
        out_specs=[
            pl.BlockSpec((1, TQ, 4 * C), lambda b, i: (b, i, 0)),
            pl.BlockSpec((1, TQ, QCD), lambda b, i: (b, i, 0)),
            pl.BlockSpec((1, TQ, WHP), lambda b, i: (b, i, 0)),
        ],
        out_shape=[
            jax.ShapeDtypeStruct((B, T, 4 * C), jnp.float32),
            jax.ShapeDtypeStruct((B, T, QCD), jnp.float32),
            jax.ShapeDtypeStruct((B, T, WHP), jnp.float32),
        ],
    )(h, Wc, W_dq, W_wp)

    from functools import partial
    c_comp, k_idx, k_all = pl.pallas_call(
        partial(_compress_body, C=C, LMAX=LMAX, PB=PB, T=T, TCHUNK=TCHUNK),
        grid=(B, P // PB),
        in_specs=[
            pl.BlockSpec((1, T, 4 * C), lambda b, p: (b, 0, 0)),
            pl.BlockSpec((1, 1, 1, PB * LMAX), lambda b, p: (b, p, 0, 0)),
            pl.BlockSpec((1, 1, 1, PB), lambda b, p: (b, p, 0, 0)),
            pl.BlockSpec((LMAX, C), lambda b, p: (0, 0)),
            pl.BlockSpec((LMAX, C), lambda b, p: (0, 0)),
            pl.BlockSpec((T, C), lambda b, p: (0, 0)),
            pl.BlockSpec((T, C), lambda b, p: (0, 0)),
            pl.BlockSpec((1, C), lambda b, p: (0, 0)),
        ],
        out_specs=[
            pl.BlockSpec((1, PB, C), lambda b, p: (b, p, 0)),
            pl.BlockSpec((1, PB, C), lambda b, p: (b, p, 0)),
            pl.BlockSpec((1, PB, C), lambda b, p: (b, p, 0)),
        ],
        out_shape=[
            jax.ShapeDtypeStruct((B, P, C), jnp.float32),
            jax.ShapeDtypeStruct((B, P, C), jnp.float32),
            jax.ShapeDtypeStruct((B, P, C), jnp.float32),
        ],
    )(pc, idx4, ep4, B_pos_kv, B_pos_idx, rope_cos, rope_sin, knw2)

    out = pl.pallas_call(
        partial(_attn_body, C=C, H=H, NIH=NIH, P=P, TT=TT, TOP_K=TOP_K),
        grid=(B, T // TT),
        in_specs=[
            pl.BlockSpec((1, TT, QCD), lambda b, i: (b, i, 0)),
            pl.BlockSpec((1, TT, WHP), lambda b, i: (b, i, 0)),
            pl.BlockSpec((1, P, C), lambda b, i: (b, 0, 0)),
            pl.BlockSpec((1, P, C), lambda b, i: (b, 0, 0)),
            pl.BlockSpec((1, P, C), lambda b, i: (b, 0, 0)),
            pl.BlockSpec((1, 1, P), lambda b, i: (b, 0, 0)),
            pl.BlockSpec((TT, C), lambda b, i: (i, 0)),
            pl.BlockSpec((TT, C), lambda b, i: (i, 0)),
            pl.BlockSpec((H * C, QCD), lambda b, i: (0, 0)),
            pl.BlockSpec((NIH * IHD, QCD), lambda b, i: (0, 0)),
            pl.BlockSpec((D, H * C), lambda b, i: (0, 0)),
            pl.BlockSpec((1, C), lambda b, i: (0, 0)),
            pl.BlockSpec((1, 128), lambda b, i: (0, 0)),
        ],
        out_specs=pl.BlockSpec((1, TT, D), lambda b, i: (b, i, 0)),
        out_shape=jax.ShapeDtypeStruct((B, T, D), jnp.float32),
    )(ql, wh, k_idx, c_comp, k_all, ep3, rope_cos, rope_sin, W_uq, W_iuq,
      W_o, qnw2, sinkp)
    return out, (pc, ql, wh, c_comp, k_idx, k_all)


def kernel(*args):
    return _phases(*args)[0]


# speedup vs baseline: 17.9332x; 1.3556x over previous
import functools
import math

import jax
import jax.numpy as jnp
from jax import lax
from jax.experimental import pallas as pl
from jax.experimental.pallas import tpu as pltpu
from jax.experimental.pallas import tpu_sc as plsc

NEG = -1e30
NEG2 = -2e30


def _bdot(a, b, dims=None):
    a = a.astype(jnp.bfloat16)
    b = b.astype(jnp.bfloat16)
    if dims is None:
        return jnp.dot(a, b, preferred_element_type=jnp.float32)
    return jax.lax.dot_general(a, b, dims, preferred_element_type=jnp.float32)


def _proj_body(h_ref, wc_ref, wdq_ref, wwp_ref, pc_ref, ql_ref, wh_ref):
    hb = h_ref[0]
    pc_ref[0] = _bdot(hb, wc_ref[...].T)
    ql_ref[0] = _bdot(hb, wdq_ref[...].T)
    wh_ref[0] = _bdot(hb, wwp_ref[...].T)


def _sc_gather_body(tab_ref, idxg_ref, cstab_ref, epg_ref,
                    g_ref, cs_ref, idx_v, rows_v, ep_v, rcs_v,
                    sem, *, NC, NW, CH, NCHUNK, RPW):
    wid = lax.axis_index("s") * NC + lax.axis_index("c")
    base = wid * CH * NCHUNK
    for c in range(NCHUNK):
        off = base + c * CH
        pltpu.sync_copy(idxg_ref.at[pl.ds(off, CH)], idx_v)
        pltpu.async_copy(tab_ref.at[idx_v], rows_v, sem).wait()
        pltpu.sync_copy(rows_v, g_ref.at[pl.ds(off, CH)])
    rbase = wid * RPW
    pltpu.sync_copy(epg_ref.at[pl.ds(rbase, RPW)], ep_v)
    pltpu.async_copy(cstab_ref.at[ep_v], rcs_v, sem).wait()
    pltpu.sync_copy(rcs_v, cs_ref.at[pl.ds(rbase, RPW)])


def _sc_gather(tab, idxg, cstab, epg):
    NTOT, W = idxg.shape[0], tab.shape[1]
    NP = epg.shape[0]
    RD = cstab.shape[1]
    info = plsc.get_sparse_core_info()
    NC, NS = info.num_cores, info.num_subcores
    NW = NC * NS
    per_w = NTOT // NW
    CH = min(128, per_w)
    NCHUNK = per_w // CH
    RPW = NP // NW
    body = functools.partial(_sc_gather_body, NC=NC, NW=NW, CH=CH,
                             NCHUNK=NCHUNK, RPW=RPW)
    mesh = plsc.VectorSubcoreMesh(core_axis_name="c", subcore_axis_name="s")
    return pl.kernel(
        body, mesh=mesh,
        out_type=[
            jax.ShapeDtypeStruct((NTOT, W), jnp.float32),
            jax.ShapeDtypeStruct((NP, RD), jnp.float32),
        ],
        scratch_types=[
            pltpu.VMEM((CH,), jnp.int32),
            pltpu.VMEM((CH, W), jnp.float32),
            pltpu.VMEM((RPW,), jnp.int32),
            pltpu.VMEM((RPW, RD), jnp.float32),
            pltpu.SemaphoreType.DMA,
        ],
    )(tab, idxg, cstab, epg)


def _compress_body(g_ref, cs_ref, bkv_ref, bidx_ref,
                   knw_ref, ccomp_ref, kidx_ref, kall_ref, *, C, LMAX, PB):
    g3 = g_ref[0].reshape(PB, LMAX, 4 * C)

    def compress(ctok, z):
        m = jnp.max(z, axis=1, keepdims=True)
        e = jnp.exp(z - m)
        gates = e / jnp.sum(e, axis=1, keepdims=True)
        return jnp.sum(gates * ctok, axis=1)

    ccomp = compress(g3[:, :, 0:C], g3[:, :, C:2 * C] + bkv_ref[...][None])
    kidx = compress(g3[:, :, 2 * C:3 * C],
                    g3[:, :, 3 * C:4 * C] + bidx_ref[...][None])
    ccomp_ref[0] = ccomp
    kidx_ref[0] = kidx

    ms = jnp.mean(ccomp * ccomp, axis=1, keepdims=True)
    kn = ccomp * jax.lax.rsqrt(ms + 1e-6) * knw_ref[...]
    cs = cs_ref[0]
    cos_k = cs[:, :C]
    sin_k = cs[:, C:]
    half = C // 2
    krot = jnp.concatenate([-kn[:, half:], kn[:, :half]], axis=1)
    kall_ref[0] = kn * cos_k + krot * sin_k


def _attn_body(ql_ref, wh_ref, kidx_ref, ccomp_ref, kall_ref, ep_ref,
               cos_ref, sin_ref, wuq_ref, wiuq_ref, wo_ref, qnw_ref,
               sinkp_ref, out_ref, *, C, H, NIH, P, TT, TOP_K):
    j = pl.program_id(1)
    ql = ql_ref[0]
    kidx = kidx_ref[0]
    ccomp = ccomp_ref[0]
    kall = kall_ref[0]
    ep = ep_ref[0]
    pos = jax.lax.broadcasted_iota(jnp.int32, (TT, 1), 0) + j * TT
    vis = ep < pos

    qi = _bdot(ql, wiuq_ref[...].T)
    wh = wh_ref[0]
    scores = jnp.zeros((TT, P), jnp.float32)
    for ih in range(NIH):
        qih = qi[:, ih * C:(ih + 1) * C]
        s = _bdot(qih, kidx.T)
        scores = scores + jnp.maximum(s, 0.0) * wh[:, ih:ih + 1]
    scores = jnp.where(vis, scores, NEG)

    lane = jax.lax.broadcasted_iota(jnp.int32, (TT, P), 1)
    work = scores
    for _ in range(TOP_K):
        m = jnp.max(work, axis=1, keepdims=True)
        j0 = jnp.min(jnp.where(work == m, lane, P), axis=1, keepdims=True)
        work = jnp.where(lane == j0, NEG2, work)
    amask = (work == NEG2) & vis
    has_any = jnp.sum(vis.astype(jnp.float32), axis=1, keepdims=True) > 0.0

    qfull = _bdot(ql, wuq_ref[...].T)
    cos_q = cos_ref[...]
    sin_q = sin_ref[...]
    half = C // 2
    qnw = qnw_ref[...]
    inv_sqrt_c = 1.0 / math.sqrt(C)
    outs = []
    for hh in range(H):
        qh = qfull[:, hh * C:(hh + 1) * C]
        ms = jnp.mean(qh * qh, axis=1, keepdims=True)
        qh = qh * jax.lax.rsqrt(ms + 1e-6) * qnw
        qrot = jnp.concatenate([-qh[:, half:], qh[:, :half]], axis=1)
        qh = qh * cos_q + qrot * sin_q
        lg = _bdot(qh, kall.T)
        lg = lg * inv_sqrt_c
        lg = jnp.where(amask, lg, NEG)
        sk = sinkp_ref[:, hh:hh + 1]
        m = jnp.maximum(jnp.max(lg, axis=1, keepdims=True), sk)
        e = jnp.exp(lg - m)
        denom = jnp.sum(e, axis=1, keepdims=True) + jnp.exp(sk - m)
        attn = e / denom
        attn = jnp.where(has_any, attn, 0.0)
        outs.append(_bdot(attn, ccomp))
    ofull = jnp.concatenate(outs, axis=1)
    out_ref[0] = _bdot(ofull, wo_ref[...].T)


def _phases(h, phrase_mask, phrase_token_idx, phrase_end_pos, rope_cos,
            rope_sin, W_dq, W_uq, W_kv_kv, W_z_kv, B_pos_kv, W_kv_idx,
            W_z_idx, B_pos_idx, W_iuq, W_w, q_norm_w, k_norm_w, W_o,
            sink_logits):
    B, T, D = h.shape
    _, P, LMAX = phrase_token_idx.shape
    C = W_kv_kv.shape[0]
    QCD = W_dq.shape[0]
    H = sink_logits.shape[0]
    IHD = W_kv_idx.shape[0]
    NIH = W_iuq.shape[0] // IHD
    TOP_K = min(32, P)
    NW = W_w.shape[0]

    TQ = min(512, T)
    PB = min(64, P)
    TT = min(256, T)

    Wc = jnp.concatenate([W_kv_kv, W_z_kv, W_kv_idx, W_z_idx], axis=0)
    WHP = 128
    W_wp = jnp.zeros((WHP, D), jnp.float32).at[:NW].set(W_w)
    ep3 = phrase_end_pos.reshape(B, 1, P).astype(jnp.int32)
    qnw2 = q_norm_w.reshape(1, C)
    knw2 = k_norm_w.reshape(1, C)
    sinkp = jnp.zeros((1, 128), jnp.float32).at[0, :H].set(sink_logits)

    pc, ql, wh = pl.pallas_call(
        _proj_body,
        grid=(B, T // TQ),
        in_specs=[
            pl.BlockSpec((1, TQ, D), lambda b, i: (b, i, 0)),
            pl.BlockSpec((4 * C, D), lambda b, i: (0, 0)),
            pl.BlockSpec((QCD, D), lambda b, i: (0, 0)),
            pl.BlockSpec((WHP, D), lambda b, i: (0, 0)),
        ],
        out_specs=[
            pl.BlockSpec((1, TQ, 4 * C), lambda b, i: (b, i, 0)),
            pl.BlockSpec((1, TQ, QCD), lambda b, i: (b, i, 0)),
            pl.BlockSpec((1, TQ, WHP), lambda b, i: (b, i, 0)),
        ],
        out_shape=[
            jax.ShapeDtypeStruct((B, T, 4 * C), jnp.float32),
            jax.ShapeDtypeStruct((B, T, QCD), jnp.float32),
            jax.ShapeDtypeStruct((B, T, WHP), jnp.float32),
        ],
    )(h, Wc, W_dq, W_wp)

    from functools import partial
    tab = pc.reshape(B * T, 4 * C)
    boff = (jnp.arange(B, dtype=jnp.int32) * T)[:, None, None]
    idxg = (phrase_token_idx.astype(jnp.int32) + boff).reshape(B * P * LMAX)
    epg = phrase_end_pos.astype(jnp.int32).reshape(B * P)
    cstab = jnp.concatenate([rope_cos, rope_sin], axis=1)
    g, cs = _sc_gather(tab, idxg, cstab, epg)
    g3 = g.reshape(B, P * LMAX, 4 * C)
    cs3 = cs.reshape(B, P, 2 * C)

    c_comp, k_idx, k_all = pl.pallas_call(
        partial(_compress_body, C=C, LMAX=LMAX, PB=PB),
        grid=(B, P // PB),
        in_specs=[
            pl.BlockSpec((1, PB * LMAX, 4 * C), lambda b, p: (b, p, 0)),
            pl.BlockSpec((1, PB, 2 * C), lambda b, p: (b, p, 0)),
            pl.BlockSpec((LMAX, C), lambda b, p: (0, 0)),
            pl.BlockSpec((LMAX, C), lambda b, p: (0, 0)),
            pl.BlockSpec((1, C), lambda b, p: (0, 0)),
        ],
        out_specs=[
            pl.BlockSpec((1, PB, C), lambda b, p: (b, p, 0)),
            pl.BlockSpec((1, PB, C), lambda b, p: (b, p, 0)),
            pl.BlockSpec((1, PB, C), lambda b, p: (b, p, 0)),
        ],
        out_shape=[
            jax.ShapeDtypeStruct((B, P, C), jnp.float32),
            jax.ShapeDtypeStruct((B, P, C), jnp.float32),
            jax.ShapeDtypeStruct((B, P, C), jnp.float32),
        ],
    )(g3, cs3, B_pos_kv, B_pos_idx, knw2)

    out = pl.pallas_call(
        partial(_attn_body, C=C, H=H, NIH=NIH, P=P, TT=TT, TOP_K=TOP_K),
        grid=(B, T // TT),
        in_specs=[
            pl.BlockSpec((1, TT, QCD), lambda b, i: (b, i, 0)),
            pl.BlockSpec((1, TT, WHP), lambda b, i: (b, i, 0)),
            pl.BlockSpec((1, P, C), lambda b, i: (b, 0, 0)),
            pl.BlockSpec((1, P, C), lambda b, i: (b, 0, 0)),
            pl.BlockSpec((1, P, C), lambda b, i: (b, 0, 0)),
            pl.BlockSpec((1, 1, P), lambda b, i: (b, 0, 0)),
            pl.BlockSpec((TT, C), lambda b, i: (i, 0)),
            pl.BlockSpec((TT, C), lambda b, i: (i, 0)),
            pl.BlockSpec((H * C, QCD), lambda b, i: (0, 0)),
            pl.BlockSpec((NIH * IHD, QCD), lambda b, i: (0, 0)),
            pl.BlockSpec((D, H * C), lambda b, i: (0, 0)),
            pl.BlockSpec((1, C), lambda b, i: (0, 0)),
            pl.BlockSpec((1, 128), lambda b, i: (0, 0)),
        ],
        out_specs=pl.BlockSpec((1, TT, D), lambda b, i: (b, i, 0)),
        out_shape=jax.ShapeDtypeStruct((B, T, D), jnp.float32),
    )(ql, wh, k_idx, c_comp, k_all, ep3, rope_cos, rope_sin, W_uq, W_iuq,
      W_o, qnw2, sinkp)
    return out, (pc, ql, wh, c_comp, k_idx, k_all)


def kernel(*args):
    return _phases(*args)[0]


# speedup vs baseline: 20.6064x; 1.1491x over previous
import functools
import math

import jax
import jax.numpy as jnp
from jax import lax
from jax.experimental import pallas as pl
from jax.experimental.pallas import tpu as pltpu
from jax.experimental.pallas import tpu_sc as plsc

NEG = -1e30
NEG2 = -2e30


def _bdot(a, b, dims=None):
    a = a.astype(jnp.bfloat16)
    b = b.astype(jnp.bfloat16)
    if dims is None:
        return jnp.dot(a, b, preferred_element_type=jnp.float32)
    return jax.lax.dot_general(a, b, dims, preferred_element_type=jnp.float32)


def _proj_body(h_ref, wc_ref, wdq_ref, wwp_ref, pc_ref, ql_ref, wh_ref):
    hb = h_ref[0]
    pc_ref[0] = _bdot(hb, wc_ref[...].T)
    ql_ref[0] = _bdot(hb, wdq_ref[...].T)
    wh_ref[0] = _bdot(hb, wwp_ref[...].T)


def _sc_gather_body(tab_ref, idxg_ref, cstab_ref, epg_ref,
                    g_ref, cs_ref, idx_v, rows_v, ep_v, rcs_v,
                    sem, *, NC, NW, CH, NCHUNK, RPW):
    wid = lax.axis_index("s") * NC + lax.axis_index("c")
    base = wid * CH * NCHUNK
    for c in range(NCHUNK):
        off = base + c * CH
        pltpu.sync_copy(idxg_ref.at[pl.ds(off, CH)], idx_v)
        pltpu.async_copy(tab_ref.at[idx_v], rows_v, sem).wait()
        pltpu.sync_copy(rows_v, g_ref.at[pl.ds(off, CH)])
    rbase = wid * RPW
    pltpu.sync_copy(epg_ref.at[pl.ds(rbase, RPW)], ep_v)
    pltpu.async_copy(cstab_ref.at[ep_v], rcs_v, sem).wait()
    pltpu.sync_copy(rcs_v, cs_ref.at[pl.ds(rbase, RPW)])


def _sc_gather(tab, idxg, cstab, epg):
    NTOT, W = idxg.shape[0], tab.shape[1]
    NP = epg.shape[0]
    RD = cstab.shape[1]
    info = plsc.get_sparse_core_info()
    NC, NS = info.num_cores, info.num_subcores
    NW = NC * NS
    per_w = NTOT // NW
    CH = min(128, per_w)
    NCHUNK = per_w // CH
    RPW = NP // NW
    body = functools.partial(_sc_gather_body, NC=NC, NW=NW, CH=CH,
                             NCHUNK=NCHUNK, RPW=RPW)
    mesh = plsc.VectorSubcoreMesh(core_axis_name="c", subcore_axis_name="s")
    return pl.kernel(
        body, mesh=mesh,
        out_type=[
            jax.ShapeDtypeStruct((NTOT, W), jnp.float32),
            jax.ShapeDtypeStruct((NP, RD), jnp.float32),
        ],
        scratch_types=[
            pltpu.VMEM((CH,), jnp.int32),
            pltpu.VMEM((CH, W), jnp.float32),
            pltpu.VMEM((RPW,), jnp.int32),
            pltpu.VMEM((RPW, RD), jnp.float32),
            pltpu.SemaphoreType.DMA,
        ],
    )(tab, idxg, cstab, epg)


def _compress_body(g_ref, cs_ref, bkv_ref, bidx_ref,
                   knw_ref, ccomp_ref, kidx_ref, kall_ref, *, C, LMAX, PB):
    g3 = g_ref[0].reshape(PB, LMAX, 4 * C)

    def compress(ctok, z):
        m = jnp.max(z, axis=1, keepdims=True)
        e = jnp.exp(z - m)
        gates = e / jnp.sum(e, axis=1, keepdims=True)
        return jnp.sum(gates * ctok, axis=1)

    ccomp = compress(g3[:, :, 0:C], g3[:, :, C:2 * C] + bkv_ref[...][None])
    kidx = compress(g3[:, :, 2 * C:3 * C],
                    g3[:, :, 3 * C:4 * C] + bidx_ref[...][None])
    ccomp_ref[0] = ccomp
    kidx_ref[0] = kidx

    ms = jnp.mean(ccomp * ccomp, axis=1, keepdims=True)
    kn = ccomp * jax.lax.rsqrt(ms + 1e-6) * knw_ref[...]
    cs = cs_ref[0]
    cos_k = cs[:, :C]
    sin_k = cs[:, C:]
    half = C // 2
    krot = jnp.concatenate([-kn[:, half:], kn[:, :half]], axis=1)
    kall_ref[0] = kn * cos_k + krot * sin_k


def _attn_body(ql_ref, wh_ref, kidx_ref, ccomp_ref, kall_ref, ep_ref,
               cos_ref, sin_ref, wuq_ref, wiuq_ref, wo_ref, qnw_ref,
               sinkp_ref, out_ref, *, C, H, NIH, P, TT, TOP_K):
    j = pl.program_id(1)
    ql = ql_ref[0]
    kidx = kidx_ref[0]
    ccomp = ccomp_ref[0]
    kall = kall_ref[0]
    ep = ep_ref[0]
    pos = jax.lax.broadcasted_iota(jnp.int32, (TT, 1), 0) + j * TT
    vis = ep < pos

    qi = _bdot(ql, wiuq_ref[...].T)
    wh = wh_ref[0]
    scores = jnp.zeros((TT, P), jnp.float32)
    for ih in range(NIH):
        qih = qi[:, ih * C:(ih + 1) * C]
        s = _bdot(qih, kidx.T)
        scores = scores + jnp.maximum(s, 0.0) * wh[:, ih:ih + 1]
    scores = jnp.where(vis, scores, NEG)

    lane = jax.lax.broadcasted_iota(jnp.int32, (TT, P), 1)
    work = scores
    for _ in range(TOP_K):
        j0 = jnp.argmax(work, axis=1)[:, None]
        work = jnp.where(lane == j0, NEG2, work)
    amask = (work == NEG2) & vis
    has_any = jnp.sum(vis.astype(jnp.float32), axis=1, keepdims=True) > 0.0

    qfull = _bdot(ql, wuq_ref[...].T)
    cos_q = cos_ref[...]
    sin_q = sin_ref[...]
    half = C // 2
    qnw = qnw_ref[...]
    inv_sqrt_c = 1.0 / math.sqrt(C)
    outs = []
    for hh in range(H):
        qh = qfull[:, hh * C:(hh + 1) * C]
        ms = jnp.mean(qh * qh, axis=1, keepdims=True)
        qh = qh * jax.lax.rsqrt(ms + 1e-6) * qnw
        qrot = jnp.concatenate([-qh[:, half:], qh[:, :half]], axis=1)
        qh = qh * cos_q + qrot * sin_q
        lg = _bdot(qh, kall.T)
        lg = lg * inv_sqrt_c
        lg = jnp.where(amask, lg, NEG)
        sk = sinkp_ref[:, hh:hh + 1]
        m = jnp.maximum(jnp.max(lg, axis=1, keepdims=True), sk)
        e = jnp.exp(lg - m)
        denom = jnp.sum(e, axis=1, keepdims=True) + jnp.exp(sk - m)
        attn = e / denom
        attn = jnp.where(has_any, attn, 0.0)
        outs.append(_bdot(attn, ccomp))
    ofull = jnp.concatenate(outs, axis=1)
    out_ref[0] = _bdot(ofull, wo_ref[...].T)


def _phases(h, phrase_mask, phrase_token_idx, phrase_end_pos, rope_cos,
            rope_sin, W_dq, W_uq, W_kv_kv, W_z_kv, B_pos_kv, W_kv_idx,
            W_z_idx, B_pos_idx, W_iuq, W_w, q_norm_w, k_norm_w, W_o,
            sink_logits):
    B, T, D = h.shape
    _, P, LMAX = phrase_token_idx.shape
    C = W_kv_kv.shape[0]
    QCD = W_dq.shape[0]
    H = sink_logits.shape[0]
    IHD = W_kv_idx.shape[0]
    NIH = W_iuq.shape[0] // IHD
    TOP_K = min(32, P)
    NW = W_w.shape[0]

    TQ = min(512, T)
    PB = min(64, P)
    TT = min(256, T)

    Wc = jnp.concatenate([W_kv_kv, W_z_kv, W_kv_idx, W_z_idx], axis=0)
    WHP = 128
    W_wp = jnp.zeros((WHP, D), jnp.float32).at[:NW].set(W_w)
    ep3 = phrase_end_pos.reshape(B, 1, P).astype(jnp.int32)
    qnw2 = q_norm_w.reshape(1, C)
    knw2 = k_norm_w.reshape(1, C)
    sinkp = jnp.zeros((1, 128), jnp.float32).at[0, :H].set(sink_logits)

    pc, ql, wh = pl.pallas_call(
        _proj_body,
        grid=(B, T // TQ),
        in_specs=[
            pl.BlockSpec((1, TQ, D), lambda b, i: (b, i, 0)),
            pl.BlockSpec((4 * C, D), lambda b, i: (0, 0)),
            pl.BlockSpec((QCD, D), lambda b, i: (0, 0)),
            pl.BlockSpec((WHP, D), lambda b, i: (0, 0)),
        ],
        out_specs=[
            pl.BlockSpec((1, TQ, 4 * C), lambda b, i: (b, i, 0)),
            pl.BlockSpec((1, TQ, QCD), lambda b, i: (b, i, 0)),
            pl.BlockSpec((1, TQ, WHP), lambda b, i: (b, i, 0)),
        ],
        out_shape=[
            jax.ShapeDtypeStruct((B, T, 4 * C), jnp.float32),
            jax.ShapeDtypeStruct((B, T, QCD), jnp.float32),
            jax.ShapeDtypeStruct((B, T, WHP), jnp.float32),
        ],
    )(h, Wc, W_dq, W_wp)

    from functools import partial
    tab = pc.reshape(B * T, 4 * C)
    boff = (jnp.arange(B, dtype=jnp.int32) * T)[:, None, None]
    idxg = (phrase_token_idx.astype(jnp.int32) + boff).reshape(B * P * LMAX)
    epg = phrase_end_pos.astype(jnp.int32).reshape(B * P)
    cstab = jnp.concatenate([rope_cos, rope_sin], axis=1)
    g, cs = _sc_gather(tab, idxg, cstab, epg)
    g3 = g.reshape(B, P * LMAX, 4 * C)
    cs3 = cs.reshape(B, P, 2 * C)

    c_comp, k_idx, k_all = pl.pallas_call(
        partial(_compress_body, C=C, LMAX=LMAX, PB=PB),
        grid=(B, P // PB),
        in_specs=[
            pl.BlockSpec((1, PB * LMAX, 4 * C), lambda b, p: (b, p, 0)),
            pl.BlockSpec((1, PB, 2 * C), lambda b, p: (b, p, 0)),
            pl.BlockSpec((LMAX, C), lambda b, p: (0, 0)),
            pl.BlockSpec((LMAX, C), lambda b, p: (0, 0)),
            pl.BlockSpec((1, C), lambda b, p: (0, 0)),
        ],
        out_specs=[
            pl.BlockSpec((1, PB, C), lambda b, p: (b, p, 0)),
            pl.BlockSpec((1, PB, C), lambda b, p: (b, p, 0)),
            pl.BlockSpec((1, PB, C), lambda b, p: (b, p, 0)),
        ],
        out_shape=[
            jax.ShapeDtypeStruct((B, P, C), jnp.float32),
            jax.ShapeDtypeStruct((B, P, C), jnp.float32),
            jax.ShapeDtypeStruct((B, P, C), jnp.float32),
        ],
    )(g3, cs3, B_pos_kv, B_pos_idx, knw2)

    out = pl.pallas_call(
        partial(_attn_body, C=C, H=H, NIH=NIH, P=P, TT=TT, TOP_K=TOP_K),
        grid=(B, T // TT),
        in_specs=[
            pl.BlockSpec((1, TT, QCD), lambda b, i: (b, i, 0)),
            pl.BlockSpec((1, TT, WHP), lambda b, i: (b, i, 0)),
            pl.BlockSpec((1, P, C), lambda b, i: (b, 0, 0)),
            pl.BlockSpec((1, P, C), lambda b, i: (b, 0, 0)),
            pl.BlockSpec((1, P, C), lambda b, i: (b, 0, 0)),
            pl.BlockSpec((1, 1, P), lambda b, i: (b, 0, 0)),
            pl.BlockSpec((TT, C), lambda b, i: (i, 0)),
            pl.BlockSpec((TT, C), lambda b, i: (i, 0)),
            pl.BlockSpec((H * C, QCD), lambda b, i: (0, 0)),
            pl.BlockSpec((NIH * IHD, QCD), lambda b, i: (0, 0)),
            pl.BlockSpec((D, H * C), lambda b, i: (0, 0)),
            pl.BlockSpec((1, C), lambda b, i: (0, 0)),
            pl.BlockSpec((1, 128), lambda b, i: (0, 0)),
        ],
        out_specs=pl.BlockSpec((1, TT, D), lambda b, i: (b, i, 0)),
        out_shape=jax.ShapeDtypeStruct((B, T, D), jnp.float32),
    )(ql, wh, k_idx, c_comp, k_all, ep3, rope_cos, rope_sin, W_uq, W_iuq,
      W_o, qnw2, sinkp)
    return out, (pc, ql, wh, c_comp, k_idx, k_all)


def kernel(*args):
    return _phases(*args)[0]


# speedup vs baseline: 21.8268x; 1.0592x over previous
import functools
import math

import jax
import jax.numpy as jnp
from jax import lax
from jax.experimental import pallas as pl
from jax.experimental.pallas import tpu as pltpu
from jax.experimental.pallas import tpu_sc as plsc

NEG = -1e30
NEG2 = -2e30


def _bdot(a, b, dims=None):
    a = a.astype(jnp.bfloat16)
    b = b.astype(jnp.bfloat16)
    if dims is None:
        return jnp.dot(a, b, preferred_element_type=jnp.float32)
    return jax.lax.dot_general(a, b, dims, preferred_element_type=jnp.float32)


def _proj_body(h_ref, wc_ref, wdq_ref, wwp_ref, pc_ref, ql_ref, wh_ref):
    hb = h_ref[0]
    pc_ref[0] = _bdot(hb, wc_ref[...].T)
    ql_ref[0] = _bdot(hb, wdq_ref[...].T)
    wh_ref[0] = _bdot(hb, wwp_ref[...].T)


def _sc_gather_body(tab_ref, idxg_ref, cstab_ref, epg_ref,
                    g_ref, cs_ref, idx_v, rows_v, ep_v, rcs_v,
                    sem, *, NC, NW, CH, NCHUNK, RPW):
    wid = lax.axis_index("s") * NC + lax.axis_index("c")
    base = wid * CH * NCHUNK
    for c in range(NCHUNK):
        off = base + c * CH
        pltpu.sync_copy(idxg_ref.at[pl.ds(off, CH)], idx_v)
        pltpu.async_copy(tab_ref.at[idx_v], rows_v, sem).wait()
        pltpu.sync_copy(rows_v, g_ref.at[pl.ds(off, CH)])
    rbase = wid * RPW
    pltpu.sync_copy(epg_ref.at[pl.ds(rbase, RPW)], ep_v)
    pltpu.async_copy(cstab_ref.at[ep_v], rcs_v, sem).wait()
    pltpu.sync_copy(rcs_v, cs_ref.at[pl.ds(rbase, RPW)])


def _sc_gather(tab, idxg, cstab, epg):
    NTOT, W = idxg.shape[0], tab.shape[1]
    NP = epg.shape[0]
    RD = cstab.shape[1]
    info = plsc.get_sparse_core_info()
    NC, NS = info.num_cores, info.num_subcores
    NW = NC * NS
    per_w = NTOT // NW
    CH = min(128, per_w)
    NCHUNK = per_w // CH
    RPW = NP // NW
    body = functools.partial(_sc_gather_body, NC=NC, NW=NW, CH=CH,
                             NCHUNK=NCHUNK, RPW=RPW)
    mesh = plsc.VectorSubcoreMesh(core_axis_name="c", subcore_axis_name="s")
    return pl.kernel(
        body, mesh=mesh,
        out_type=[
            jax.ShapeDtypeStruct((NTOT, W), jnp.float32),
            jax.ShapeDtypeStruct((NP, RD), jnp.float32),
        ],
        scratch_types=[
            pltpu.VMEM((CH,), jnp.int32),
            pltpu.VMEM((CH, W), jnp.float32),
            pltpu.VMEM((RPW,), jnp.int32),
            pltpu.VMEM((RPW, RD), jnp.float32),
            pltpu.SemaphoreType.DMA,
        ],
    )(tab, idxg, cstab, epg)


def _compress_body(g_ref, cs_ref, bkv_ref, bidx_ref,
                   knw_ref, ccomp_ref, kidx_ref, kall_ref, *, C, LMAX, PB):
    g3 = g_ref[0].reshape(PB, LMAX, 4 * C)

    def compress(ctok, z):
        m = jnp.max(z, axis=1, keepdims=True)
        e = jnp.exp(z - m)
        gates = e / jnp.sum(e, axis=1, keepdims=True)
        return jnp.sum(gates * ctok, axis=1)

    ccomp = compress(g3[:, :, 0:C], g3[:, :, C:2 * C] + bkv_ref[...][None])
    kidx = compress(g3[:, :, 2 * C:3 * C],
                    g3[:, :, 3 * C:4 * C] + bidx_ref[...][None])
    ccomp_ref[0] = ccomp
    kidx_ref[0] = kidx

    ms = jnp.mean(ccomp * ccomp, axis=1, keepdims=True)
    kn = ccomp * jax.lax.rsqrt(ms + 1e-6) * knw_ref[...]
    cs = cs_ref[0]
    cos_k = cs[:, :C]
    sin_k = cs[:, C:]
    half = C // 2
    krot = jnp.concatenate([-kn[:, half:], kn[:, :half]], axis=1)
    kall_ref[0] = kn * cos_k + krot * sin_k


def _attn_body(ql_ref, wh_ref, kidx_ref, ccomp_ref, kall_ref, ep_ref,
               cos_ref, sin_ref, wuq_ref, wiuq_ref, wo_ref, qnw_ref,
               sinkp_ref, out_ref, *, C, H, NIH, P, TT, TOP_K):
    j = pl.program_id(1)
    ql = ql_ref[0]
    kidx = kidx_ref[0]
    ccomp = ccomp_ref[0]
    kall = kall_ref[0]
    ep = ep_ref[0]
    pos = jax.lax.broadcasted_iota(jnp.int32, (TT, 1), 0) + j * TT
    vis = ep < pos

    qi = _bdot(ql, wiuq_ref[...].T)
    wh = wh_ref[0]
    scores = jnp.zeros((TT, P), jnp.float32)
    for ih in range(NIH):
        qih = qi[:, ih * C:(ih + 1) * C]
        s = _bdot(qih, kidx.T)
        scores = scores + jnp.maximum(s, 0.0) * wh[:, ih:ih + 1]
    scores = jnp.where(vis, scores, NEG)

    lane = jax.lax.broadcasted_iota(jnp.int32, (TT, P), 1)
    work = scores
    for _ in range(TOP_K):
        m = jnp.max(work, axis=1, keepdims=True)
        j0 = jnp.min(jnp.where(work == m, lane, P), axis=1, keepdims=True)
        work = jnp.where(lane == j0, NEG2, work)
    amask = (work == NEG2) & vis
    has_any = jnp.sum(vis.astype(jnp.float32), axis=1, keepdims=True) > 0.0

    qfull = _bdot(ql, wuq_ref[...].T)
    cos_q = cos_ref[...]
    sin_q = sin_ref[...]
    half = C // 2
    qnw = qnw_ref[...]
    inv_sqrt_c = 1.0 / math.sqrt(C)
    outs = []
    for hh in range(H):
        qh = qfull[:, hh * C:(hh + 1) * C]
        ms = jnp.mean(qh * qh, axis=1, keepdims=True)
        qh = qh * jax.lax.rsqrt(ms + 1e-6) * qnw
        qrot = jnp.concatenate([-qh[:, half:], qh[:, :half]], axis=1)
        qh = qh * cos_q + qrot * sin_q
        lg = _bdot(qh, kall.T)
        lg = lg * inv_sqrt_c
        lg = jnp.where(amask, lg, NEG)
        sk = sinkp_ref[:, hh:hh + 1]
        m = jnp.maximum(jnp.max(lg, axis=1, keepdims=True), sk)
        e = jnp.exp(lg - m)
        denom = jnp.sum(e, axis=1, keepdims=True) + jnp.exp(sk - m)
        attn = e / denom
        attn = jnp.where(has_any, attn, 0.0)
        outs.append(_bdot(attn, ccomp))
    ofull = jnp.concatenate(outs, axis=1)
    out_ref[0] = _bdot(ofull, wo_ref[...].T)


def _phases(h, phrase_mask, phrase_token_idx, phrase_end_pos, rope_cos,
            rope_sin, W_dq, W_uq, W_kv_kv, W_z_kv, B_pos_kv, W_kv_idx,
            W_z_idx, B_pos_idx, W_iuq, W_w, q_norm_w, k_norm_w, W_o,
            sink_logits):
    B, T, D = h.shape
    _, P, LMAX = phrase_token_idx.shape
    C = W_kv_kv.shape[0]
    QCD = W_dq.shape[0]
    H = sink_logits.shape[0]
    IHD = W_kv_idx.shape[0]
    NIH = W_iuq.shape[0] // IHD
    TOP_K = min(32, P)
    NW = W_w.shape[0]

    TQ = min(512, T)
    PB = min(64, P)
    TT = min(512, T)

    Wc = jnp.concatenate([W_kv_kv, W_z_kv, W_kv_idx, W_z_idx], axis=0)
    WHP = 128
    W_wp = jnp.zeros((WHP, D), jnp.float32).at[:NW].set(W_w)
    ep3 = phrase_end_pos.reshape(B, 1, P).astype(jnp.int32)
    qnw2 = q_norm_w.reshape(1, C)
    knw2 = k_norm_w.reshape(1, C)
    sinkp = jnp.zeros((1, 128), jnp.float32).at[0, :H].set(sink_logits)

    pc, ql, wh = pl.pallas_call(
        _proj_body,
        grid=(B, T // TQ),
        in_specs=[
            pl.BlockSpec((1, TQ, D), lambda b, i: (b, i, 0)),
            pl.BlockSpec((4 * C, D), lambda b, i: (0, 0)),
            pl.BlockSpec((QCD, D), lambda b, i: (0, 0)),
            pl.BlockSpec((WHP, D), lambda b, i: (0, 0)),
        ],
        out_specs=[
            pl.BlockSpec((1, TQ, 4 * C), lambda b, i: (b, i, 0)),
            pl.BlockSpec((1, TQ, QCD), lambda b, i: (b, i, 0)),
            pl.BlockSpec((1, TQ, WHP), lambda b, i: (b, i, 0)),
        ],
        out_shape=[
            jax.ShapeDtypeStruct((B, T, 4 * C), jnp.float32),
            jax.ShapeDtypeStruct((B, T, QCD), jnp.float32),
            jax.ShapeDtypeStruct((B, T, WHP), jnp.float32),
        ],
    )(h, Wc, W_dq, W_wp)

    from functools import partial
    tab = pc.reshape(B * T, 4 * C)
    boff = (jnp.arange(B, dtype=jnp.int32) * T)[:, None, None]
    idxg = (phrase_token_idx.astype(jnp.int32) + boff).reshape(B * P * LMAX)
    epg = phrase_end_pos.astype(jnp.int32).reshape(B * P)
    cstab = jnp.concatenate([rope_cos, rope_sin], axis=1)
    g, cs = _sc_gather(tab, idxg, cstab, epg)
    g3 = g.reshape(B, P * LMAX, 4 * C)
    cs3 = cs.reshape(B, P, 2 * C)

    c_comp, k_idx, k_all = pl.pallas_call(
        partial(_compress_body, C=C, LMAX=LMAX, PB=PB),
        grid=(B, P // PB),
        in_specs=[
            pl.BlockSpec((1, PB * LMAX, 4 * C), lambda b, p: (b, p, 0)),
            pl.BlockSpec((1, PB, 2 * C), lambda b, p: (b, p, 0)),
            pl.BlockSpec((LMAX, C), lambda b, p: (0, 0)),
            pl.BlockSpec((LMAX, C), lambda b, p: (0, 0)),
            pl.BlockSpec((1, C), lambda b, p: (0, 0)),
        ],
        out_specs=[
            pl.BlockSpec((1, PB, C), lambda b, p: (b, p, 0)),
            pl.BlockSpec((1, PB, C), lambda b, p: (b, p, 0)),
            pl.BlockSpec((1, PB, C), lambda b, p: (b, p, 0)),
        ],
        out_shape=[
            jax.ShapeDtypeStruct((B, P, C), jnp.float32),
            jax.ShapeDtypeStruct((B, P, C), jnp.float32),
            jax.ShapeDtypeStruct((B, P, C), jnp.float32),
        ],
    )(g3, cs3, B_pos_kv, B_pos_idx, knw2)

    out = pl.pallas_call(
        partial(_attn_body, C=C, H=H, NIH=NIH, P=P, TT=TT, TOP_K=TOP_K),
        grid=(B, T // TT),
        in_specs=[
            pl.BlockSpec((1, TT, QCD), lambda b, i: (b, i, 0)),
            pl.BlockSpec((1, TT, WHP), lambda b, i: (b, i, 0)),
            pl.BlockSpec((1, P, C), lambda b, i: (b, 0, 0)),
            pl.BlockSpec((1, P, C), lambda b, i: (b, 0, 0)),
            pl.BlockSpec((1, P, C), lambda b, i: (b, 0, 0)),
            pl.BlockSpec((1, 1, P), lambda b, i: (b, 0, 0)),
            pl.BlockSpec((TT, C), lambda b, i: (i, 0)),
            pl.BlockSpec((TT, C), lambda b, i: (i, 0)),
            pl.BlockSpec((H * C, QCD), lambda b, i: (0, 0)),
            pl.BlockSpec((NIH * IHD, QCD), lambda b, i: (0, 0)),
            pl.BlockSpec((D, H * C), lambda b, i: (0, 0)),
            pl.BlockSpec((1, C), lambda b, i: (0, 0)),
            pl.BlockSpec((1, 128), lambda b, i: (0, 0)),
        ],
        out_specs=pl.BlockSpec((1, TT, D), lambda b, i: (b, i, 0)),
        out_shape=jax.ShapeDtypeStruct((B, T, D), jnp.float32),
    )(ql, wh, k_idx, c_comp, k_all, ep3, rope_cos, rope_sin, W_uq, W_iuq,
      W_o, qnw2, sinkp)
    return out, (pc, ql, wh, c_comp, k_idx, k_all)


def kernel(*args):
    return _phases(*args)[0]
